# baseline (device time: 54347 ns/iter reference)
import os

import jax
import jax.numpy as jnp
from jax import lax
from jax.experimental import pallas as pl
from jax.experimental.pallas import tpu as pltpu

_SKIP_COMM = os.environ.get("SKIP_COMM") == "1"
_SKIP_COMPUTE = os.environ.get("SKIP_COMPUTE") == "1"

N_DEV = 32
B, SQ, SKV, DH = 2, 256, 256, 64
HL = 4
DM = 512
HCOLS = HL * DH
ROWS = B * SQ
CH = ROWS // N_DEV
NP = N_DEV - 1


def _v_from_l(l):
    z = l >> 3
    p = l & 7
    y = p >> 1
    x = (p & 1) ^ (y & 1)
    return (
        (x << 4) | ((y & 1) << 3) | ((z & 1) << 2)
        | ((y >> 1) << 1) | (z >> 1)
    )


def _l_from_v(v):
    x = (v >> 4) & 1
    y = (((v >> 1) & 1) << 1) | ((v >> 3) & 1)
    z = ((v & 1) << 1) | ((v >> 2) & 1)
    p = (y << 1) | (x ^ (y & 1))
    return (z << 3) | p


def kernel(x, Wq, K_ext, V_ext, Wo):
    i = lax.axis_index("i")
    Wq_l = lax.dynamic_slice(Wq, (0, i * HCOLS), (DM, HCOLS))
    Wo_l = lax.dynamic_slice(Wo, (i * HCOLS, 0), (HCOLS, DM))

    def body(x_ref, wq_ref, k_ref, v_ref, wo_ref, out_ref,
             acc, rbuf, rs_send, rs_recv, ag_send, ag_recv):
        me = lax.axis_index("i")
        v = _v_from_l(me)

        maskf = (
            jnp.abs(
                lax.broadcasted_iota(jnp.int32, (SQ, SKV), 0)
                - lax.broadcasted_iota(jnp.int32, (SQ, SKV), 1)
            )
            <= 128
        ).astype(jnp.float32)
        if _SKIP_COMPUTE:
            acc[:, :] = x_ref[0, :, :] + x_ref[1, :, :]
        else:
            x2 = jnp.reshape(x_ref[:, :, :], (ROWS, DM))
            q2 = jnp.dot(x2, wq_ref[:, :], preferred_element_type=jnp.float32)
            ctx_rows = []
            for b in range(B):
                ctx_cols = []
                for h in range(HL):
                    q_h = q2[b * SQ:(b + 1) * SQ, h * DH:(h + 1) * DH]
                    k_h = k_ref[b, :, h, :]
                    v_h = v_ref[b, :, h, :]
                    s = lax.dot_general(
                        q_h, k_h, (((1,), (1,)), ((), ())),
                        preferred_element_type=jnp.float32,
                    ) * 0.125
                    e = jnp.exp(s) * maskf
                    denom = jnp.sum(e, axis=-1, keepdims=True)
                    ctx_cols.append(
                        jnp.dot(e, v_h, preferred_element_type=jnp.float32)
                        / denom
                    )
                ctx_rows.append(jnp.concatenate(ctx_cols, axis=1))
            ctx2 = jnp.concatenate(ctx_rows, axis=0)
            acc[:, :] = jnp.dot(
                ctx2, wo_ref[:, :], preferred_element_type=jnp.float32
            )

        if _SKIP_COMM:
            out_ref[0, :, :] = acc[0:SQ, :]
            out_ref[1, :, :] = acc[SQ:ROWS, :]
            return

        rs_descs = []
        for t in range(N_DEV):
            slot = (v ^ t) - 1
            rdma = pltpu.make_async_remote_copy(
                src_ref=acc.at[pl.ds(t * CH, CH)],
                dst_ref=rbuf.at[pl.ds(slot * CH, CH)],
                send_sem=rs_send.at[slot],
                recv_sem=rs_recv.at[slot],
                device_id=(_l_from_v(t),),
                device_id_type=pl.DeviceIdType.MESH,
            )
            rs_descs.append(rdma)

            @pl.when(t != v)
            def _(rdma=rdma):
                rdma.start()

        for t in range(N_DEV):
            @pl.when(t != v)
            def _(rdma=rs_descs[t]):
                rdma.wait_recv()

        mine = acc[pl.ds(v * CH, CH), :]
        others = jnp.sum(
            jnp.reshape(rbuf[0:NP * CH, :], (NP, CH, DM)), axis=0
        )
        acc[pl.ds(v * CH, CH), :] = mine + others

        ag_descs = []
        for d in range(1, N_DEV):
            peer = _l_from_v(v ^ d)
            rdma = pltpu.make_async_remote_copy(
                src_ref=acc.at[pl.ds(v * CH, CH)],
                dst_ref=acc.at[pl.ds(v * CH, CH)],
                send_sem=ag_send.at[d - 1],
                recv_sem=ag_recv.at[d - 1],
                device_id=(peer,),
                device_id_type=pl.DeviceIdType.MESH,
            )
            rdma.start()
            ag_descs.append(rdma)
        for rdma in ag_descs:
            rdma.wait_recv()

        out_ref[0, :, :] = acc[0:SQ, :]
        out_ref[1, :, :] = acc[SQ:ROWS, :]

        for t in range(N_DEV):
            @pl.when(t != v)
            def _(rdma=rs_descs[t]):
                rdma.wait_send()
        for rdma in ag_descs:
            rdma.wait_send()

    return pl.pallas_call(
        body,
        out_shape=jax.ShapeDtypeStruct((B, SQ, DM), jnp.float32),
        in_specs=[pl.BlockSpec(memory_space=pltpu.VMEM)] * 5,
        out_specs=pl.BlockSpec(memory_space=pltpu.VMEM),
        scratch_shapes=[
            pltpu.VMEM((ROWS, DM), jnp.float32),
            pltpu.VMEM((NP * CH, DM), jnp.float32),
            pltpu.SemaphoreType.DMA((NP,)),
            pltpu.SemaphoreType.DMA((NP,)),
            pltpu.SemaphoreType.DMA((NP,)),
            pltpu.SemaphoreType.DMA((NP,)),
        ],
    )(x, Wq_l, K_ext, V_ext, Wo_l)


# device time: 52038 ns/iter; 1.0444x vs baseline; 1.0444x over previous
import os

import jax
import jax.numpy as jnp
from jax import lax
from jax.experimental import pallas as pl
from jax.experimental.pallas import tpu as pltpu

_SKIP_COMM = os.environ.get("SKIP_COMM") == "1"
_SKIP_COMPUTE = os.environ.get("SKIP_COMPUTE") == "1"

N_DEV = 32
B, SQ, SKV, DH = 2, 256, 256, 64
HL = 4
DM = 512
HCOLS = HL * DH
ROWS = B * SQ
CH = ROWS // N_DEV
NP = N_DEV - 1


def _v_from_l(l):
    z = l >> 3
    p = l & 7
    y = p >> 1
    x = (p & 1) ^ (y & 1)
    return (
        (x << 4) | ((y & 1) << 3) | ((z & 1) << 2)
        | ((y >> 1) << 1) | (z >> 1)
    )


def _l_from_v(v):
    x = (v >> 4) & 1
    y = (((v >> 1) & 1) << 1) | ((v >> 3) & 1)
    z = ((v & 1) << 1) | ((v >> 2) & 1)
    p = (y << 1) | (x ^ (y & 1))
    return (z << 3) | p


def kernel(x, Wq, K_ext, V_ext, Wo):
    i = lax.axis_index("i")
    Wq_l = lax.dynamic_slice(Wq, (0, i * HCOLS), (DM, HCOLS))
    Wo_l = lax.dynamic_slice(Wo, (i * HCOLS, 0), (HCOLS, DM))

    def body(x_ref, wq_ref, k_ref, v_ref, wo_ref, out_ref,
             acc, rbuf, rs_send, rs_recv, ag_send, ag_recv):
        me = lax.axis_index("i")
        v = _v_from_l(me)

        maskf = (
            jnp.abs(
                lax.broadcasted_iota(jnp.int32, (SQ, SKV), 0)
                - lax.broadcasted_iota(jnp.int32, (SQ, SKV), 1)
            )
            <= 128
        ).astype(jnp.float32)
        if _SKIP_COMPUTE:
            acc[:, :] = jnp.reshape(x_ref[:, :, :], (ROWS, DM))
        else:
            x2 = jnp.reshape(x_ref[:, :, :], (ROWS, DM))
            q2 = jnp.dot(x2, wq_ref[:, :], preferred_element_type=jnp.float32)
            ctx_rows = []
            for b in range(B):
                ctx_cols = []
                for h in range(HL):
                    q_h = q2[b * SQ:(b + 1) * SQ, h * DH:(h + 1) * DH]
                    k_h = k_ref[b, :, h, :]
                    v_h = v_ref[b, :, h, :]
                    s = lax.dot_general(
                        q_h, k_h, (((1,), (1,)), ((), ())),
                        preferred_element_type=jnp.float32,
                    ) * 0.125
                    e = jnp.exp(s) * maskf
                    denom = jnp.sum(e, axis=-1, keepdims=True)
                    ctx_cols.append(
                        jnp.dot(e, v_h, preferred_element_type=jnp.float32)
                        / denom
                    )
                ctx_rows.append(jnp.concatenate(ctx_cols, axis=1))
            ctx2 = jnp.concatenate(ctx_rows, axis=0)
            acc[:, :] = jnp.dot(
                ctx2, wo_ref[:, :], preferred_element_type=jnp.float32
            )

        if _SKIP_COMM:
            out_ref[0, :, :] = acc[0:SQ, :]
            out_ref[1, :, :] = acc[SQ:ROWS, :]
            return

        rs_descs = []
        for t in range(N_DEV):
            slot = (v ^ t) - 1
            rdma = pltpu.make_async_remote_copy(
                src_ref=acc.at[pl.ds(t * CH, CH)],
                dst_ref=rbuf.at[pl.ds(slot * CH, CH)],
                send_sem=rs_send.at[slot],
                recv_sem=rs_recv.at[slot],
                device_id=(_l_from_v(t),),
                device_id_type=pl.DeviceIdType.MESH,
            )
            rs_descs.append(rdma)

            @pl.when(t != v)
            def _(rdma=rdma):
                rdma.start()

        for t in range(N_DEV):
            @pl.when(t != v)
            def _(rdma=rs_descs[t]):
                rdma.wait_recv()

        mine = acc[pl.ds(v * CH, CH), :]
        others = jnp.sum(
            jnp.reshape(rbuf[0:NP * CH, :], (NP, CH, DM)), axis=0
        )
        acc[pl.ds(v * CH, CH), :] = mine + others

        ag_descs = []
        for d in range(1, N_DEV):
            peer = _l_from_v(v ^ d)
            rdma = pltpu.make_async_remote_copy(
                src_ref=acc.at[pl.ds(v * CH, CH)],
                dst_ref=acc.at[pl.ds(v * CH, CH)],
                send_sem=ag_send.at[d - 1],
                recv_sem=ag_recv.at[d - 1],
                device_id=(peer,),
                device_id_type=pl.DeviceIdType.MESH,
            )
            rdma.start()
            ag_descs.append(rdma)
        for rdma in ag_descs:
            rdma.wait_recv()

        out_ref[0, :, :] = acc[0:SQ, :]
        out_ref[1, :, :] = acc[SQ:ROWS, :]

        for t in range(N_DEV):
            @pl.when(t != v)
            def _(rdma=rs_descs[t]):
                rdma.wait_send()
        for rdma in ag_descs:
            rdma.wait_send()

    return pl.pallas_call(
        body,
        out_shape=jax.ShapeDtypeStruct((B, SQ, DM), jnp.float32),
        in_specs=[pl.BlockSpec(memory_space=pltpu.VMEM)] * 5,
        out_specs=pl.BlockSpec(memory_space=pltpu.VMEM),
        scratch_shapes=[
            pltpu.VMEM((ROWS, DM), jnp.float32),
            pltpu.VMEM((NP * CH, DM), jnp.float32),
            pltpu.SemaphoreType.DMA((NP,)),
            pltpu.SemaphoreType.DMA((NP,)),
            pltpu.SemaphoreType.DMA((NP,)),
            pltpu.SemaphoreType.DMA((NP,)),
        ],
    )(x, Wq_l, K_ext, V_ext, Wo_l)
